# Initial kernel scaffold; baseline (speedup 1.0000x reference)
import jax
import jax.numpy as jnp
from jax.experimental import pallas as pl


def kernel(x, edge_index, Wq1, bq1, Wk1, bk1, Wv1, bv1, Ws1, bs1, Wq2, bq2, Wk2, bk2, Wv2, bv2, Ws2, bs2):
    raise NotImplementedError("write your pallas kernel here")


# speedup vs baseline: 1.1484x; 1.1484x over previous
import functools
import math

import jax
import jax.numpy as jnp
from jax.experimental import pallas as pl

_N = 10000
_H = 4
_C = 128
_MB = 1000


def _mm_kernel(x_ref, w_ref, b_ref, o_ref):
    o_ref[...] = (
        jnp.dot(x_ref[...], w_ref[...], preferred_element_type=jnp.float32)
        + b_ref[...]
    )


def _project(x, w, b):
    n, k = x.shape
    m = w.shape[1]
    grid = n // _MB
    return pl.pallas_call(
        _mm_kernel,
        grid=(grid,),
        in_specs=[
            pl.BlockSpec((_MB, k), lambda i: (i, 0)),
            pl.BlockSpec((k, m), lambda i: (0, 0)),
            pl.BlockSpec((1, m), lambda i: (0, 0)),
        ],
        out_specs=pl.BlockSpec((_MB, m), lambda i: (i, 0)),
        out_shape=jax.ShapeDtypeStruct((n, m), jnp.float32),
    )(x, w, b)


def _edge_phase(q, k, v, src, dst, heads):
    n = q.shape[0]
    c = q.shape[1] // heads
    qe = q[dst].reshape(-1, heads, c)
    ke = k[src].reshape(-1, heads, c)
    ve = v[src].reshape(-1, heads, c)
    alpha = (qe * ke).sum(-1) * (1.0 / math.sqrt(c))
    p = jnp.exp(alpha)
    s = jax.ops.segment_sum(p, dst, num_segments=n)
    num = jax.ops.segment_sum(p[:, :, None] * ve, dst, num_segments=n)
    out = num / (s[:, :, None] + 1e-16)
    return out.reshape(n, heads * c)


def kernel(x, edge_index, Wq1, bq1, Wk1, bk1, Wv1, bv1, Ws1, bs1,
           Wq2, bq2, Wk2, bk2, Wv2, bv2, Ws2, bs2):
    src = edge_index[0]
    dst = edge_index[1]

    w1 = jnp.concatenate([Wq1, Wk1, Wv1, Ws1], axis=1)
    b1 = jnp.concatenate([bq1, bk1, bv1, bs1])[None, :]
    t1 = _project(x, w1, b1)
    q1, k1, v1, s1 = (t1[:, 0:512], t1[:, 512:1024], t1[:, 1024:1536],
                      t1[:, 1536:2048])
    h1 = jax.nn.relu(_edge_phase(q1, k1, v1, src, dst, _H) + s1)

    w2 = jnp.concatenate([Wq2, Wk2, Wv2, Ws2], axis=1)
    b2 = jnp.concatenate([bq2, bk2, bv2, bs2])[None, :]
    t2 = _project(h1, w2, b2)
    q2, k2, v2, s2 = (t2[:, 0:128], t2[:, 128:256], t2[:, 256:384],
                      t2[:, 384:512])
    out = _edge_phase(q2, k2, v2, src, dst, 1) + s2
    return out
